# baseline (speedup 1.0000x reference)
import functools

import jax
import jax.numpy as jnp
from jax import lax
from jax.experimental import pallas as pl
from jax.experimental.pallas import tpu as pltpu
from jax.experimental.pallas import tpu_sc as plsc

NUM_CORES = 2
NUM_SUBCORES = 16
NUM_WORKERS = NUM_CORES * NUM_SUBCORES
LANES = 16

BATCH = 16384
DIM = 64
B_PER_W = BATCH // NUM_WORKERS


def _mf_body(user_hbm, item_hbm, uemb_hbm, iemb_hbm, ubias_hbm, ibias_hbm,
             gbias_hbm, out_hbm,
             uidx_v, iidx_v, urows_v, irows_v, ub_v, ib_v, out_v, part_v,
             gb_v, sem):
    wid = lax.axis_index("s") * NUM_CORES + lax.axis_index("c")
    base = wid * B_PER_W

    pltpu.sync_copy(user_hbm.at[pl.ds(base, B_PER_W)], uidx_v)
    pltpu.sync_copy(item_hbm.at[pl.ds(base, B_PER_W)], iidx_v)
    pltpu.sync_copy(gbias_hbm, gb_v.at[pl.ds(0, 1)])

    cu = pltpu.async_copy(uemb_hbm.at[uidx_v], urows_v, sem)
    ci = pltpu.async_copy(iemb_hbm.at[iidx_v], irows_v, sem)
    cbu = pltpu.async_copy(ubias_hbm.at[uidx_v], ub_v, sem)
    cbi = pltpu.async_copy(ibias_hbm.at[iidx_v], ib_v, sem)
    cu.wait()
    ci.wait()
    cbu.wait()
    cbi.wait()

    gb = gb_v[...][0]
    lane_iota = lax.iota(jnp.int32, LANES)

    def group_body(g, carry):
        base_row = g * LANES
        for r16 in range(LANES):
            urow = urows_v.at[base_row + r16]
            irow = irows_v.at[base_row + r16]
            s = urow[pl.ds(0, LANES)] * irow[pl.ds(0, LANES)]
            for c in range(1, DIM // LANES):
                s = s + urow[pl.ds(c * LANES, LANES)] * irow[pl.ds(c * LANES, LANES)]
            part_v[pl.ds(r16 * LANES, LANES)] = s
        bu = ub_v[pl.ds(base_row, LANES)]
        bi = ib_v[pl.ds(base_row, LANES)]
        acc = bu + bi + gb
        for c in range(LANES):
            acc = acc + plsc.load_gather(part_v, [lane_iota * LANES + c])
        out_v[pl.ds(base_row, LANES)] = acc
        return carry

    lax.fori_loop(0, B_PER_W // LANES, group_body, 0)

    pltpu.sync_copy(out_v, out_hbm.at[pl.ds(base, B_PER_W)])


_mf_kernel = functools.partial(
    pl.kernel,
    out_type=jax.ShapeDtypeStruct((BATCH,), jnp.float32),
    mesh=plsc.VectorSubcoreMesh(core_axis_name="c", subcore_axis_name="s",
                                num_cores=NUM_CORES,
                                num_subcores=NUM_SUBCORES),
    scratch_types=[
        pltpu.VMEM((B_PER_W,), jnp.int32),
        pltpu.VMEM((B_PER_W,), jnp.int32),
        pltpu.VMEM((B_PER_W, DIM), jnp.float32),
        pltpu.VMEM((B_PER_W, DIM), jnp.float32),
        pltpu.VMEM((B_PER_W,), jnp.float32),
        pltpu.VMEM((B_PER_W,), jnp.float32),
        pltpu.VMEM((B_PER_W,), jnp.float32),
        pltpu.VMEM((LANES * LANES,), jnp.float32),
        pltpu.VMEM((LANES,), jnp.float32),
        pltpu.SemaphoreType.DMA,
    ],
    compiler_params=pltpu.CompilerParams(needs_layout_passes=False,
                                         use_tc_tiling_on_sc=False),
)(_mf_body)


@jax.jit
def kernel(user, item, user_emb, item_emb, user_bias, item_bias, global_bias):
    user = user.astype(jnp.int32)
    item = item.astype(jnp.int32)
    return _mf_kernel(user, item, user_emb, item_emb,
                      user_bias.reshape(-1), item_bias.reshape(-1),
                      global_bias)


# speedup vs baseline: 1.3234x; 1.3234x over previous
import functools

import jax
import jax.numpy as jnp
from jax import lax
from jax.experimental import pallas as pl
from jax.experimental.pallas import tpu as pltpu
from jax.experimental.pallas import tpu_sc as plsc

NUM_CORES = 2
NUM_SUBCORES = 16
NUM_WORKERS = NUM_CORES * NUM_SUBCORES
LANES = 16

BATCH = 16384
DIM = 64
SUB = 8
B_PER_W = BATCH // NUM_WORKERS
CHUNK = 16
N_CHUNKS = B_PER_W // CHUNK
BUF_ROWS = CHUNK * SUB


def _mf_body(user_hbm, item_hbm, uemb_hbm, iemb_hbm, ubias_hbm, ibias_hbm,
             gbias_hbm, out_hbm,
             uidx_v, iidx_v, ur0, ir0, ur1, ir1, ub_v, ib_v, out_v, part_v,
             gb_v, sem0, sem1, semb):
    wid = lax.axis_index("s") * NUM_CORES + lax.axis_index("c")
    base = wid * B_PER_W

    pltpu.sync_copy(user_hbm.at[pl.ds(base, B_PER_W)], uidx_v)
    pltpu.sync_copy(item_hbm.at[pl.ds(base, B_PER_W)], iidx_v)
    pltpu.sync_copy(gbias_hbm, gb_v.at[pl.ds(0, 1)])

    row_bufs = [(ur0, ir0), (ur1, ir1)]
    sems = [sem0, sem1]

    def fire_bias(g, carry):
        u_vec = uidx_v[pl.ds(g * LANES, LANES)]
        i_vec = iidx_v[pl.ds(g * LANES, LANES)]
        ublk = (u_vec // SUB) * SUB
        iblk = (i_vec // SUB) * SUB
        for r in range(LANES):
            j = g * LANES + r
            ub = pl.multiple_of(ublk[r], SUB)
            ib = pl.multiple_of(iblk[r], SUB)
            pltpu.async_copy(ubias_hbm.at[pl.ds(ub, SUB)],
                             ub_v.at[pl.ds(j * SUB, SUB)], semb)
            pltpu.async_copy(ibias_hbm.at[pl.ds(ib, SUB)],
                             ib_v.at[pl.ds(j * SUB, SUB)], semb)
        return carry

    lax.fori_loop(0, B_PER_W // LANES, fire_bias, 0)

    def fire_chunk(c, parity):
        urb, irb = row_bufs[parity]
        sem = sems[parity]
        u_vec = uidx_v[pl.ds(c * CHUNK, CHUNK)]
        i_vec = iidx_v[pl.ds(c * CHUNK, CHUNK)]
        ublk = (u_vec // SUB) * SUB
        iblk = (i_vec // SUB) * SUB
        for r in range(CHUNK):
            ub = pl.multiple_of(ublk[r], SUB)
            ib = pl.multiple_of(iblk[r], SUB)
            pltpu.async_copy(uemb_hbm.at[pl.ds(ub, SUB), :],
                             urb.at[pl.ds(r * SUB, SUB), :], sem)
            pltpu.async_copy(iemb_hbm.at[pl.ds(ib, SUB), :],
                             irb.at[pl.ds(r * SUB, SUB), :], sem)

    def drain_chunk(parity):
        urb, irb = row_bufs[parity]
        sem = sems[parity]
        pltpu.make_async_copy(uemb_hbm.at[pl.ds(0, BUF_ROWS), :], urb,
                              sem).wait()
        pltpu.make_async_copy(iemb_hbm.at[pl.ds(0, BUF_ROWS), :], irb,
                              sem).wait()

    def drain_bias():
        pltpu.make_async_copy(out_hbm.at[pl.ds(0, B_PER_W * SUB)], ub_v,
                              semb).wait()
        pltpu.make_async_copy(out_hbm.at[pl.ds(0, B_PER_W * SUB)], ib_v,
                              semb).wait()

    lane_iota = lax.iota(jnp.int32, LANES)

    def compute_chunk(c, parity):
        urb, irb = row_bufs[parity]
        cb = c * CHUNK
        gb = gb_v[...][0]
        u_vec = uidx_v[pl.ds(cb, CHUNK)]
        i_vec = iidx_v[pl.ds(cb, CHUNK)]
        u_sub = u_vec % SUB
        i_sub = i_vec % SUB
        for r in range(CHUNK):
            urow = urb.at[r * SUB + u_sub[r]]
            irow = irb.at[r * SUB + i_sub[r]]
            s = urow[pl.ds(0, LANES)] * irow[pl.ds(0, LANES)]
            for cc in range(1, DIM // LANES):
                s = s + (urow[pl.ds(cc * LANES, LANES)]
                         * irow[pl.ds(cc * LANES, LANES)])
            part_v[pl.ds(r * LANES, LANES)] = s
        slot_base = (lane_iota + cb) * SUB
        bu = plsc.load_gather(ub_v, [slot_base + u_sub])
        bi = plsc.load_gather(ib_v, [slot_base + i_sub])
        acc = bu + bi + gb
        for cc in range(LANES):
            acc = acc + plsc.load_gather(part_v, [lane_iota * LANES + cc])
        out_v[pl.ds(cb, LANES)] = acc

    fire_chunk(0, 0)
    fire_chunk(1, 1)
    drain_bias()

    def pipeline_body(k, carry):
        c0 = 2 * k
        drain_chunk(0)
        compute_chunk(c0, 0)
        fire_chunk(c0 + 2, 0)
        drain_chunk(1)
        compute_chunk(c0 + 1, 1)
        fire_chunk(c0 + 3, 1)
        return carry

    lax.fori_loop(0, N_CHUNKS // 2 - 1, pipeline_body, 0)
    drain_chunk(0)
    compute_chunk(N_CHUNKS - 2, 0)
    drain_chunk(1)
    compute_chunk(N_CHUNKS - 1, 1)

    pltpu.sync_copy(out_v, out_hbm.at[pl.ds(base, B_PER_W)])


_mf_kernel = functools.partial(
    pl.kernel,
    out_type=jax.ShapeDtypeStruct((BATCH,), jnp.float32),
    mesh=plsc.VectorSubcoreMesh(core_axis_name="c", subcore_axis_name="s",
                                num_cores=NUM_CORES,
                                num_subcores=NUM_SUBCORES),
    scratch_types=[
        pltpu.VMEM((B_PER_W,), jnp.int32),
        pltpu.VMEM((B_PER_W,), jnp.int32),
        pltpu.VMEM((BUF_ROWS, DIM), jnp.float32),
        pltpu.VMEM((BUF_ROWS, DIM), jnp.float32),
        pltpu.VMEM((BUF_ROWS, DIM), jnp.float32),
        pltpu.VMEM((BUF_ROWS, DIM), jnp.float32),
        pltpu.VMEM((B_PER_W * SUB,), jnp.float32),
        pltpu.VMEM((B_PER_W * SUB,), jnp.float32),
        pltpu.VMEM((B_PER_W,), jnp.float32),
        pltpu.VMEM((LANES * LANES,), jnp.float32),
        pltpu.VMEM((LANES,), jnp.float32),
        pltpu.SemaphoreType.DMA,
        pltpu.SemaphoreType.DMA,
        pltpu.SemaphoreType.DMA,
    ],
    compiler_params=pltpu.CompilerParams(needs_layout_passes=False,
                                         use_tc_tiling_on_sc=True),
)(_mf_body)


@jax.jit
def kernel(user, item, user_emb, item_emb, user_bias, item_bias, global_bias):
    user = user.astype(jnp.int32)
    item = item.astype(jnp.int32)
    return _mf_kernel(user, item, user_emb, item_emb,
                      user_bias.reshape(-1), item_bias.reshape(-1),
                      global_bias)
